# baseline (speedup 1.0000x reference)
import functools

import jax
import jax.numpy as jnp
from jax import lax
from jax.experimental import pallas as pl
from jax.experimental.pallas import tpu as pltpu
from jax.experimental.pallas import tpu_sc as plsc

N = 10000
D = 128
E = 320000

NC = 2
NS = 16
EPT = E // (NC * NS)
CHUNK = 128
NCHUNK = -(-EPT // CHUNK)
PAD = NCHUNK * CHUNK - EPT
RPT = N // NS
ACC_ROWS = N + 8


def _spmm_body(h_hbm, src_hbm, dst_hbm, zeros_hbm, out_hbm,
               acc, sidx, didx, rows, sem):
    c = lax.axis_index("c")
    s = lax.axis_index("s")

    ic = pltpu.async_copy(src_hbm.at[c, s], sidx, sem)
    pltpu.sync_copy(dst_hbm.at[c, s], didx)
    pltpu.sync_copy(zeros_hbm, acc.at[pl.ds(s * RPT, RPT)])
    ic.wait()

    plsc.subcore_barrier()

    def body(j, carry):
        pltpu.async_copy(h_hbm.at[sidx.at[j]], rows, sem).wait()
        pltpu.sync_copy(rows, acc.at[didx.at[j]], add=True)
        return carry

    lax.fori_loop(0, NCHUNK, body, 0)

    plsc.subcore_barrier()

    pltpu.sync_copy(acc.at[pl.ds(s * RPT, RPT)], out_hbm.at[c, s])


_spmm = functools.partial(
    pl.kernel,
    out_type=jax.ShapeDtypeStruct((NC, NS, RPT, D), jnp.float32),
    mesh=plsc.VectorSubcoreMesh(core_axis_name="c", subcore_axis_name="s"),
    scratch_types=[
        pltpu.VMEM_SHARED((ACC_ROWS, D), jnp.float32),
        pltpu.VMEM((NCHUNK, CHUNK), jnp.int32),
        pltpu.VMEM((NCHUNK, CHUNK), jnp.int32),
        pltpu.VMEM((CHUNK, D), jnp.float32),
        pltpu.SemaphoreType.DMA,
    ],
)(_spmm_body)


def _mm1_body(x_ref, w_ref, b_ref, o_ref):
    o_ref[...] = (
        jnp.dot(x_ref[...], w_ref[...], preferred_element_type=jnp.float32)
        + b_ref[...]
    )


def _mm2_body(p0_ref, p1_ref, w_ref, b_ref, o_ref):
    h = p0_ref[...] + p1_ref[...]
    o_ref[...] = (
        jnp.dot(h, w_ref[...], preferred_element_type=jnp.float32) + b_ref[...]
    )


def _add_body(a_ref, b_ref, o_ref):
    o_ref[...] = a_ref[...] + b_ref[...]


_ROWS_BLK = 1000
_GRID = N // _ROWS_BLK

_row_spec = pl.BlockSpec((_ROWS_BLK, D), lambda i: (i, 0))
_w_spec = pl.BlockSpec((D, D), lambda i: (0, 0))
_b_spec = pl.BlockSpec((1, D), lambda i: (0, 0))

_mm1 = pl.pallas_call(
    _mm1_body,
    grid=(_GRID,),
    in_specs=[_row_spec, _w_spec, _b_spec],
    out_specs=_row_spec,
    out_shape=jax.ShapeDtypeStruct((N, D), jnp.float32),
)

_mm2 = pl.pallas_call(
    _mm2_body,
    grid=(_GRID,),
    in_specs=[_row_spec, _row_spec, _w_spec, _b_spec],
    out_specs=_row_spec,
    out_shape=jax.ShapeDtypeStruct((N, D), jnp.float32),
)

_add = pl.pallas_call(
    _add_body,
    grid=(_GRID,),
    in_specs=[_row_spec, _row_spec],
    out_specs=_row_spec,
    out_shape=jax.ShapeDtypeStruct((N, D), jnp.float32),
)


def kernel(x, adj, W1, b1, W2, b2):
    src = adj[0].astype(jnp.int32).reshape(NC * NS, EPT)
    dst = adj[1].astype(jnp.int32).reshape(NC * NS, EPT)
    src = jnp.pad(src, ((0, 0), (0, PAD)))
    dst = jnp.pad(dst, ((0, 0), (0, PAD)), constant_values=N)
    src = src.reshape(NC, NS, NCHUNK, CHUNK)
    dst = dst.reshape(NC, NS, NCHUNK, CHUNK)
    zeros = jnp.zeros((RPT, D), jnp.float32)

    h = _mm1(x, W1, b1.reshape(1, D))
    p = _spmm(h, src, dst, zeros).reshape(NC, N, D)
    h = _mm2(p[0], p[1], W2, b2.reshape(1, D))
    q = _spmm(h, src, dst, zeros).reshape(NC, N, D)
    return _add(q[0], q[1])


# speedup vs baseline: 1.3866x; 1.3866x over previous
import functools

import jax
import jax.numpy as jnp
from jax import lax
from jax.experimental import pallas as pl
from jax.experimental.pallas import tpu as pltpu
from jax.experimental.pallas import tpu_sc as plsc

N = 10000
D = 128
E = 320000

NC = 2
NS = 16
EPT = E // (NC * NS)
CHUNK = 80
NCHUNK = -(-EPT // CHUNK)
PAD = NCHUNK * CHUNK - EPT
RPT = N // NS
ACC_ROWS = N + 8


def _spmm_body(h_hbm, src_hbm, dst_hbm, zeros_hbm, out_hbm,
               acc, sidx, didx, rows, sem):
    c = lax.axis_index("c")
    s = lax.axis_index("s")

    ic = pltpu.async_copy(src_hbm.at[c, s], sidx, sem)
    pltpu.sync_copy(dst_hbm.at[c, s], didx)
    pltpu.sync_copy(zeros_hbm, acc.at[pl.ds(s * RPT, RPT)])
    ic.wait()

    plsc.subcore_barrier()

    def body(j, carry):
        pltpu.async_copy(h_hbm.at[sidx.at[j]], rows, sem).wait()
        pltpu.sync_copy(rows, acc.at[didx.at[j]], add=True)
        return carry

    lax.fori_loop(0, NCHUNK, body, 0)

    plsc.subcore_barrier()

    pltpu.sync_copy(acc.at[pl.ds(s * RPT, RPT)], out_hbm.at[c, s])


_spmm = functools.partial(
    pl.kernel,
    out_type=jax.ShapeDtypeStruct((NC, NS, RPT, D), jnp.float32),
    mesh=plsc.VectorSubcoreMesh(core_axis_name="c", subcore_axis_name="s"),
    scratch_types=[
        pltpu.VMEM_SHARED((ACC_ROWS, D), jnp.float32),
        pltpu.VMEM((NCHUNK, CHUNK), jnp.int32),
        pltpu.VMEM((NCHUNK, CHUNK), jnp.int32),
        pltpu.VMEM((CHUNK, D), jnp.float32),
        pltpu.SemaphoreType.DMA,
    ],
)(_spmm_body)


def _mm1_body(x_ref, w_ref, b_ref, o_ref):
    o_ref[...] = (
        jnp.dot(x_ref[...], w_ref[...], preferred_element_type=jnp.float32)
        + b_ref[...]
    )


def _mm2_body(p0_ref, p1_ref, w_ref, b_ref, o_ref):
    h = p0_ref[...] + p1_ref[...]
    o_ref[...] = (
        jnp.dot(h, w_ref[...], preferred_element_type=jnp.float32) + b_ref[...]
    )


def _add_body(a_ref, b_ref, o_ref):
    o_ref[...] = a_ref[...] + b_ref[...]


_ROWS_BLK = 1000
_GRID = N // _ROWS_BLK

_row_spec = pl.BlockSpec((_ROWS_BLK, D), lambda i: (i, 0))
_w_spec = pl.BlockSpec((D, D), lambda i: (0, 0))
_b_spec = pl.BlockSpec((1, D), lambda i: (0, 0))

_mm1 = pl.pallas_call(
    _mm1_body,
    grid=(_GRID,),
    in_specs=[_row_spec, _w_spec, _b_spec],
    out_specs=_row_spec,
    out_shape=jax.ShapeDtypeStruct((N, D), jnp.float32),
)

_mm2 = pl.pallas_call(
    _mm2_body,
    grid=(_GRID,),
    in_specs=[_row_spec, _row_spec, _w_spec, _b_spec],
    out_specs=_row_spec,
    out_shape=jax.ShapeDtypeStruct((N, D), jnp.float32),
)

_add = pl.pallas_call(
    _add_body,
    grid=(_GRID,),
    in_specs=[_row_spec, _row_spec],
    out_specs=_row_spec,
    out_shape=jax.ShapeDtypeStruct((N, D), jnp.float32),
)


def kernel(x, adj, W1, b1, W2, b2):
    src = adj[0].astype(jnp.int32).reshape(NC * NS, EPT)
    dst = adj[1].astype(jnp.int32).reshape(NC * NS, EPT)
    src = jnp.pad(src, ((0, 0), (0, PAD)))
    dst = jnp.pad(dst, ((0, 0), (0, PAD)), constant_values=N)
    src = src.reshape(NC, NS, NCHUNK, CHUNK)
    dst = dst.reshape(NC, NS, NCHUNK, CHUNK)
    zeros = jnp.zeros((RPT, D), jnp.float32)

    h = _mm1(x, W1, b1.reshape(1, D))
    p = _spmm(h, src, dst, zeros).reshape(NC, N, D)
    h = _mm2(p[0], p[1], W2, b2.reshape(1, D))
    q = _spmm(h, src, dst, zeros).reshape(NC, N, D)
    return _add(q[0], q[1])
